# baseline (speedup 1.0000x reference)
import functools

import jax
import jax.numpy as jnp
from jax import lax
from jax.experimental import pallas as pl
from jax.experimental.pallas import tpu as pltpu
from jax.experimental.pallas import tpu_sc as plsc

N = 10000
E = 320000
H = 128
P = 16384

NC = 2
NS = 16
NW = NC * NS
CH = 128
G = 80
EW = G * CH
E_PAD = NW * EW
ACC_ROWS = 10112
RPT = ACC_ROWS // NS

_mesh = plsc.VectorSubcoreMesh(core_axis_name="c", subcore_axis_name="s")
_HIGH = jax.lax.Precision.HIGHEST


def _worker_id():
    return lax.axis_index("s") * NC + lax.axis_index("c")


@functools.partial(
    pl.kernel,
    out_type=jax.ShapeDtypeStruct((NC, ACC_ROWS, H), jnp.float32),
    mesh=_mesh,
    scratch_types=[
        pltpu.VMEM((G, CH), jnp.int32),
        pltpu.VMEM((CH, H), jnp.float32),
        pltpu.VMEM_SHARED((ACC_ROWS, H), jnp.float32),
    ],
)
def _deg(dstw, ones, zeros, out, dst_v, ones_v, acc):
    c = lax.axis_index("c")
    s = lax.axis_index("s")
    w = _worker_id()
    pltpu.sync_copy(dstw.at[w], dst_v)
    pltpu.sync_copy(ones, ones_v)
    pltpu.sync_copy(zeros.at[pl.ds(s * RPT, RPT)], acc.at[pl.ds(s * RPT, RPT)])
    plsc.subcore_barrier()

    def body(g, _):
        pltpu.sync_copy(ones_v, acc.at[dst_v.at[g]], add=True)
        return 0

    lax.fori_loop(0, G, body, 0)
    plsc.subcore_barrier()
    pltpu.sync_copy(acc.at[pl.ds(s * RPT, RPT)], out.at[c].at[pl.ds(s * RPT, RPT)])


SB = 5
GS = G // SB


@functools.partial(
    pl.kernel,
    out_type=jax.ShapeDtypeStruct((NC, ACC_ROWS, H), jnp.float32),
    mesh=_mesh,
    scratch_types=[
        pltpu.VMEM((GS, CH), jnp.int32),
        pltpu.VMEM((GS, CH), jnp.int32),
        pltpu.VMEM((CH, H), jnp.float32),
        pltpu.VMEM((CH, H), jnp.float32),
        pltpu.VMEM_SHARED((ACC_ROWS, H), jnp.float32),
        pltpu.SemaphoreType.DMA,
        pltpu.SemaphoreType.DMA,
        pltpu.SemaphoreType.DMA,
        pltpu.SemaphoreType.DMA,
    ],
)
def _agg(hp, srcw, dstw, zeros, out, src_v, dst_v, buf_a, buf_b, acc,
         gsem_a, gsem_b, ssem_a, ssem_b):
    c = lax.axis_index("c")
    s = lax.axis_index("s")
    w = _worker_id()
    pltpu.sync_copy(zeros.at[pl.ds(s * RPT, RPT)], acc.at[pl.ds(s * RPT, RPT)])
    plsc.subcore_barrier()

    def superblock(sb, _):
        pltpu.sync_copy(srcw.at[w, pl.ds(sb * GS, GS)], src_v)
        pltpu.sync_copy(dstw.at[w, pl.ds(sb * GS, GS)], dst_v)
        pltpu.async_copy(hp.at[src_v.at[0]], buf_a, gsem_a)
        pltpu.async_copy(hp.at[src_v.at[1]], buf_b, gsem_b)

        def body(i, _):
            g0 = 2 * i
            pltpu.make_async_copy(hp.at[src_v.at[g0]], buf_a, gsem_a).wait()
            pltpu.async_copy(buf_a, acc.at[dst_v.at[g0]], ssem_a, add=True)
            pltpu.make_async_copy(hp.at[src_v.at[g0 + 1]], buf_b, gsem_b).wait()
            pltpu.async_copy(buf_b, acc.at[dst_v.at[g0 + 1]], ssem_b, add=True)
            pltpu.make_async_copy(buf_a, acc.at[dst_v.at[g0]], ssem_a).wait()
            pltpu.async_copy(hp.at[src_v.at[g0 + 2]], buf_a, gsem_a)
            pltpu.make_async_copy(buf_b, acc.at[dst_v.at[g0 + 1]], ssem_b).wait()
            pltpu.async_copy(hp.at[src_v.at[g0 + 3]], buf_b, gsem_b)
            return 0

        lax.fori_loop(0, GS // 2 - 1, body, 0)
        g0 = GS - 2
        pltpu.make_async_copy(hp.at[src_v.at[g0]], buf_a, gsem_a).wait()
        pltpu.async_copy(buf_a, acc.at[dst_v.at[g0]], ssem_a, add=True)
        pltpu.make_async_copy(hp.at[src_v.at[g0 + 1]], buf_b, gsem_b).wait()
        pltpu.async_copy(buf_b, acc.at[dst_v.at[g0 + 1]], ssem_b, add=True)
        pltpu.make_async_copy(buf_a, acc.at[dst_v.at[g0]], ssem_a).wait()
        pltpu.make_async_copy(buf_b, acc.at[dst_v.at[g0 + 1]], ssem_b).wait()
        return 0

    lax.fori_loop(0, SB, superblock, 0)
    plsc.subcore_barrier()
    pltpu.sync_copy(acc.at[pl.ds(s * RPT, RPT)], out.at[c].at[pl.ds(s * RPT, RPT)])


PG = (2 * P) // NW // CH


@functools.partial(
    pl.kernel,
    out_type=jax.ShapeDtypeStruct((2 * P, H), jnp.float32),
    mesh=_mesh,
    scratch_types=[
        pltpu.VMEM((PG, CH), jnp.int32),
        pltpu.VMEM((CH, H), jnp.float32),
        pltpu.VMEM((CH, H), jnp.float32),
        pltpu.SemaphoreType.DMA,
        pltpu.SemaphoreType.DMA,
    ],
)
def _pairs(h3, idxw, out, idx_v, buf_a, buf_b, sem_a, sem_b):
    w = _worker_id()
    base = w * (PG * CH)
    pltpu.sync_copy(idxw.at[w], idx_v)
    bufs = (buf_a, buf_b)
    sems = (sem_a, sem_b)
    pltpu.async_copy(h3.at[idx_v.at[0]], buf_a, sem_a)
    pltpu.async_copy(h3.at[idx_v.at[1]], buf_b, sem_b)
    for g in range(PG):
        b, sm = bufs[g % 2], sems[g % 2]
        pltpu.make_async_copy(h3.at[idx_v.at[g]], b, sm).wait()
        pltpu.sync_copy(b, out.at[pl.ds(base + g * CH, CH)])
        if g + 2 < PG:
            pltpu.async_copy(h3.at[idx_v.at[g + 2]], b, sm)


BR = 2000
GRID_N = N // BR


def _scale_body(dp_ref, dis_ref, dinv_ref):
    deg = dp_ref[0, :, 0:1] + dp_ref[1, :, 0:1] + 1.0
    dis_ref[...] = lax.rsqrt(deg)
    dinv_ref[...] = 1.0 / deg


def _pre_body(x_ref, w_ref, dis_ref, hw_ref, hp_ref):
    hw = jnp.dot(x_ref[...], w_ref[...], precision=_HIGH,
                 preferred_element_type=jnp.float32)
    hw_ref[...] = hw
    hp_ref[...] = hw * dis_ref[...]


def _mid_body(s_ref, hw_ref, dis_ref, dinv_ref, b_ref, w_ref, hwo_ref, hpo_ref):
    dis = dis_ref[...]
    t = (dis * (s_ref[0] + s_ref[1]) + dinv_ref[...] * hw_ref[...] + b_ref[...])
    t = jnp.maximum(t, 0.0)
    hw2 = jnp.dot(t, w_ref[...], precision=_HIGH,
                  preferred_element_type=jnp.float32)
    hwo_ref[...] = hw2
    hpo_ref[...] = hw2 * dis


def _fin_body(s_ref, hw_ref, dis_ref, dinv_ref, b_ref, h3_ref):
    h3_ref[...] = (dis_ref[...] * (s_ref[0] + s_ref[1])
                   + dinv_ref[...] * hw_ref[...] + b_ref[...])


BP = 512
GRID_P = P // BP


def _pred_body(gs_ref, gd_ref, wa_ref, wb_ref, bp1_ref, wp2_ref, bp2_ref, out_ref):
    e = (jnp.dot(gs_ref[...], wa_ref[...], precision=_HIGH,
                 preferred_element_type=jnp.float32)
         + jnp.dot(gd_ref[...], wb_ref[...], precision=_HIGH,
                   preferred_element_type=jnp.float32)
         + bp1_ref[...])
    e = jnp.maximum(e, 0.0)
    z = jnp.sum(e * wp2_ref[...], axis=1, keepdims=True) + bp2_ref[...]
    out_ref[...] = 1.0 / (1.0 + jnp.exp(-z))


def _row_spec(shape):
    return pl.BlockSpec(shape, lambda i: (i, 0))


def _full_spec(shape):
    return pl.BlockSpec(shape, lambda i: (0, 0))


_s_spec = pl.BlockSpec((NC, BR, H), lambda i: (0, i, 0))

_tc_scale = pl.pallas_call(
    _scale_body,
    grid=(GRID_N,),
    in_specs=[_s_spec],
    out_specs=[_row_spec((BR, 1)), _row_spec((BR, 1))],
    out_shape=[jax.ShapeDtypeStruct((N, 1), jnp.float32)] * 2,
)

_tc_pre = pl.pallas_call(
    _pre_body,
    grid=(GRID_N,),
    in_specs=[_row_spec((BR, H)), _full_spec((H, H)), _row_spec((BR, 1))],
    out_specs=[_row_spec((BR, H)), _row_spec((BR, H))],
    out_shape=[jax.ShapeDtypeStruct((N, H), jnp.float32)] * 2,
)

_tc_mid = pl.pallas_call(
    _mid_body,
    grid=(GRID_N,),
    in_specs=[_s_spec, _row_spec((BR, H)), _row_spec((BR, 1)),
              _row_spec((BR, 1)), _full_spec((1, H)), _full_spec((H, H))],
    out_specs=[_row_spec((BR, H)), _row_spec((BR, H))],
    out_shape=[jax.ShapeDtypeStruct((N, H), jnp.float32)] * 2,
)

_tc_fin = pl.pallas_call(
    _fin_body,
    grid=(GRID_N,),
    in_specs=[_s_spec, _row_spec((BR, H)), _row_spec((BR, 1)),
              _row_spec((BR, 1)), _full_spec((1, H))],
    out_specs=_row_spec((BR, H)),
    out_shape=jax.ShapeDtypeStruct((N, H), jnp.float32),
)

_tc_pred = pl.pallas_call(
    _pred_body,
    grid=(GRID_P,),
    in_specs=[_row_spec((BP, H)), _row_spec((BP, H)), _full_spec((H, H)),
              _full_spec((H, H)), _full_spec((1, H)), _full_spec((1, H)),
              _full_spec((1, 1))],
    out_specs=_row_spec((BP, 1)),
    out_shape=jax.ShapeDtypeStruct((P, 1), jnp.float32),
)


def kernel(x, edge_index, edge_pairs, W1, b1, W2, b2, W3, b3, Wp1, bp1, Wp2, bp2):
    src = edge_index[0]
    dst = edge_index[1]
    npad = E_PAD - E
    pad_i = jnp.arange(npad, dtype=jnp.int32)
    srcw = jnp.concatenate([src, (pad_i * 997) % N]).reshape(NW, G, CH)
    dstw = jnp.concatenate([dst, N + (pad_i % 16)]).reshape(NW, G, CH)
    idxw = jnp.concatenate([edge_pairs[0], edge_pairs[1]]).reshape(NW, PG, CH)

    zeros = jnp.zeros((ACC_ROWS, H), jnp.float32)
    ones = jnp.ones((CH, H), jnp.float32)

    deg_parts = _deg(dstw, ones, zeros)
    dis, dinv = _tc_scale(deg_parts)

    hw1, hp1 = _tc_pre(x, W1, dis)
    s1 = _agg(hp1, srcw, dstw, zeros)
    hw2, hp2 = _tc_mid(s1, hw1, dis, dinv, b1.reshape(1, H), W2)
    s2 = _agg(hp2, srcw, dstw, zeros)
    hw3, hp3 = _tc_mid(s2, hw2, dis, dinv, b2.reshape(1, H), W3)
    s3 = _agg(hp3, srcw, dstw, zeros)
    h3 = _tc_fin(s3, hw3, dis, dinv, b3.reshape(1, H))

    g = _pairs(h3, idxw)
    pred = _tc_pred(g[:P], g[P:], Wp1[:H], Wp1[H:], bp1.reshape(1, H),
                    Wp2.reshape(1, H), bp2.reshape(1, 1))
    return pred.reshape(P)


# speedup vs baseline: 1.1807x; 1.1807x over previous
import functools

import jax
import jax.numpy as jnp
from jax import lax
from jax.experimental import pallas as pl
from jax.experimental.pallas import tpu as pltpu
from jax.experimental.pallas import tpu_sc as plsc

N = 10000
E = 320000
H = 128
P = 16384

NC = 2
NS = 16
NW = NC * NS
CH = 128
G = 80
EW = G * CH
E_PAD = NW * EW
ACC_ROWS = 10112
RPT = ACC_ROWS // NS

_mesh = plsc.VectorSubcoreMesh(core_axis_name="c", subcore_axis_name="s")
_HIGH = jax.lax.Precision.HIGHEST


def _worker_id():
    return lax.axis_index("s") * NC + lax.axis_index("c")


@functools.partial(
    pl.kernel,
    out_type=jax.ShapeDtypeStruct((NC, ACC_ROWS, H), jnp.float32),
    mesh=_mesh,
    scratch_types=[
        pltpu.VMEM((G, CH), jnp.int32),
        pltpu.VMEM((CH, H), jnp.float32),
        pltpu.VMEM_SHARED((ACC_ROWS, H), jnp.float32),
    ],
)
def _deg(dstw, ones, zeros, out, dst_v, ones_v, acc):
    c = lax.axis_index("c")
    s = lax.axis_index("s")
    w = _worker_id()
    pltpu.sync_copy(dstw.at[w], dst_v)
    pltpu.sync_copy(ones, ones_v)
    pltpu.sync_copy(zeros.at[pl.ds(s * RPT, RPT)], acc.at[pl.ds(s * RPT, RPT)])
    plsc.subcore_barrier()

    def body(g, _):
        pltpu.sync_copy(ones_v, acc.at[dst_v.at[g]], add=True)
        return 0

    lax.fori_loop(0, G, body, 0)
    plsc.subcore_barrier()
    pltpu.sync_copy(acc.at[pl.ds(s * RPT, RPT)], out.at[c].at[pl.ds(s * RPT, RPT)])


SB = 5
GS = G // SB


@functools.partial(
    pl.kernel,
    out_type=jax.ShapeDtypeStruct((NC, ACC_ROWS, H), jnp.float32),
    mesh=_mesh,
    scratch_types=[
        pltpu.VMEM((GS, CH), jnp.int32),
        pltpu.VMEM((GS, CH), jnp.int32),
        pltpu.VMEM((CH, H), jnp.float32),
        pltpu.VMEM((CH, H), jnp.float32),
        pltpu.VMEM_SHARED((ACC_ROWS, H), jnp.float32),
        pltpu.SemaphoreType.DMA,
        pltpu.SemaphoreType.DMA,
    ],
)
def _agg(hp, srcw, dstw, zeros, out, src_v, dst_v, buf_a, buf_b, acc,
         gsem_a, gsem_b):
    c = lax.axis_index("c")
    s = lax.axis_index("s")
    w = _worker_id()
    pltpu.sync_copy(zeros.at[pl.ds(s * RPT, RPT)], acc.at[pl.ds(s * RPT, RPT)])
    plsc.subcore_barrier()

    def superblock(sb, _):
        pltpu.sync_copy(srcw.at[w, pl.ds(sb * GS, GS)], src_v)
        pltpu.sync_copy(dstw.at[w, pl.ds(sb * GS, GS)], dst_v)
        pltpu.async_copy(hp.at[src_v.at[0]], buf_a, gsem_a)
        pltpu.async_copy(hp.at[src_v.at[1]], buf_b, gsem_b)

        def body(i, _):
            g0 = 2 * i
            pltpu.make_async_copy(hp.at[src_v.at[g0]], buf_a, gsem_a).wait()
            pltpu.sync_copy(buf_a, acc.at[dst_v.at[g0]], add=True)
            pltpu.async_copy(hp.at[src_v.at[g0 + 2]], buf_a, gsem_a)
            pltpu.make_async_copy(hp.at[src_v.at[g0 + 1]], buf_b, gsem_b).wait()
            pltpu.sync_copy(buf_b, acc.at[dst_v.at[g0 + 1]], add=True)
            pltpu.async_copy(hp.at[src_v.at[g0 + 3]], buf_b, gsem_b)
            return 0

        lax.fori_loop(0, GS // 2 - 1, body, 0)
        pltpu.make_async_copy(hp.at[src_v.at[GS - 2]], buf_a, gsem_a).wait()
        pltpu.sync_copy(buf_a, acc.at[dst_v.at[GS - 2]], add=True)
        pltpu.make_async_copy(hp.at[src_v.at[GS - 1]], buf_b, gsem_b).wait()
        pltpu.sync_copy(buf_b, acc.at[dst_v.at[GS - 1]], add=True)
        return 0

    lax.fori_loop(0, SB, superblock, 0)
    plsc.subcore_barrier()
    pltpu.sync_copy(acc.at[pl.ds(s * RPT, RPT)], out.at[c].at[pl.ds(s * RPT, RPT)])


PG = (2 * P) // NW // CH


@functools.partial(
    pl.kernel,
    out_type=jax.ShapeDtypeStruct((2 * P, H), jnp.float32),
    mesh=_mesh,
    scratch_types=[
        pltpu.VMEM((PG, CH), jnp.int32),
        pltpu.VMEM((CH, H), jnp.float32),
        pltpu.VMEM((CH, H), jnp.float32),
        pltpu.SemaphoreType.DMA,
        pltpu.SemaphoreType.DMA,
    ],
)
def _pairs(h3, idxw, out, idx_v, buf_a, buf_b, sem_a, sem_b):
    w = _worker_id()
    base = w * (PG * CH)
    pltpu.sync_copy(idxw.at[w], idx_v)
    bufs = (buf_a, buf_b)
    sems = (sem_a, sem_b)
    pltpu.async_copy(h3.at[idx_v.at[0]], buf_a, sem_a)
    pltpu.async_copy(h3.at[idx_v.at[1]], buf_b, sem_b)
    for g in range(PG):
        b, sm = bufs[g % 2], sems[g % 2]
        pltpu.make_async_copy(h3.at[idx_v.at[g]], b, sm).wait()
        pltpu.sync_copy(b, out.at[pl.ds(base + g * CH, CH)])
        if g + 2 < PG:
            pltpu.async_copy(h3.at[idx_v.at[g + 2]], b, sm)


BR = 2000
GRID_N = N // BR


def _mm_body(x_ref, w_ref, hw_ref):
    hw_ref[...] = jnp.dot(x_ref[...], w_ref[...], precision=_HIGH,
                          preferred_element_type=jnp.float32)


def _sp_body(dp_ref, hw_ref, dis_ref, dinv_ref, hp_ref):
    deg = dp_ref[0, :, 0:1] + dp_ref[1, :, 0:1] + 1.0
    dis = lax.rsqrt(deg)
    dis_ref[...] = dis
    dinv_ref[...] = 1.0 / deg
    hp_ref[...] = hw_ref[...] * dis


def _mid_body(s_ref, hw_ref, dis_ref, dinv_ref, b_ref, w_ref, hwo_ref, hpo_ref):
    dis = dis_ref[...]
    t = (dis * (s_ref[0] + s_ref[1]) + dinv_ref[...] * hw_ref[...] + b_ref[...])
    t = jnp.maximum(t, 0.0)
    hw2 = jnp.dot(t, w_ref[...], precision=_HIGH,
                  preferred_element_type=jnp.float32)
    hwo_ref[...] = hw2
    hpo_ref[...] = hw2 * dis


def _fin_body(s_ref, hw_ref, dis_ref, dinv_ref, b_ref, h3_ref):
    h3_ref[...] = (dis_ref[...] * (s_ref[0] + s_ref[1])
                   + dinv_ref[...] * hw_ref[...] + b_ref[...])


BP = 512
GRID_P = P // BP


def _pred_body(gs_ref, gd_ref, wa_ref, wb_ref, bp1_ref, wp2_ref, bp2_ref, out_ref):
    e = (jnp.dot(gs_ref[...], wa_ref[...], precision=_HIGH,
                 preferred_element_type=jnp.float32)
         + jnp.dot(gd_ref[...], wb_ref[...], precision=_HIGH,
                   preferred_element_type=jnp.float32)
         + bp1_ref[...])
    e = jnp.maximum(e, 0.0)
    z = jnp.sum(e * wp2_ref[...], axis=1, keepdims=True) + bp2_ref[...]
    out_ref[...] = 1.0 / (1.0 + jnp.exp(-z))


def _row_spec(shape):
    return pl.BlockSpec(shape, lambda i: (i, 0))


def _full_spec(shape):
    return pl.BlockSpec(shape, lambda i: (0, 0))


_s_spec = pl.BlockSpec((NC, BR, H), lambda i: (0, i, 0))

_tc_mm = pl.pallas_call(
    _mm_body,
    grid=(GRID_N,),
    in_specs=[_row_spec((BR, H)), _full_spec((H, H))],
    out_specs=_row_spec((BR, H)),
    out_shape=jax.ShapeDtypeStruct((N, H), jnp.float32),
)

_tc_sp = pl.pallas_call(
    _sp_body,
    grid=(GRID_N,),
    in_specs=[_s_spec, _row_spec((BR, H))],
    out_specs=[_row_spec((BR, 1)), _row_spec((BR, 1)), _row_spec((BR, H))],
    out_shape=[jax.ShapeDtypeStruct((N, 1), jnp.float32),
               jax.ShapeDtypeStruct((N, 1), jnp.float32),
               jax.ShapeDtypeStruct((N, H), jnp.float32)],
)

_tc_mid = pl.pallas_call(
    _mid_body,
    grid=(GRID_N,),
    in_specs=[_s_spec, _row_spec((BR, H)), _row_spec((BR, 1)),
              _row_spec((BR, 1)), _full_spec((1, H)), _full_spec((H, H))],
    out_specs=[_row_spec((BR, H)), _row_spec((BR, H))],
    out_shape=[jax.ShapeDtypeStruct((N, H), jnp.float32)] * 2,
)

_tc_fin = pl.pallas_call(
    _fin_body,
    grid=(GRID_N,),
    in_specs=[_s_spec, _row_spec((BR, H)), _row_spec((BR, 1)),
              _row_spec((BR, 1)), _full_spec((1, H))],
    out_specs=_row_spec((BR, H)),
    out_shape=jax.ShapeDtypeStruct((N, H), jnp.float32),
)

_tc_pred = pl.pallas_call(
    _pred_body,
    grid=(GRID_P,),
    in_specs=[_row_spec((BP, H)), _row_spec((BP, H)), _full_spec((H, H)),
              _full_spec((H, H)), _full_spec((1, H)), _full_spec((1, H)),
              _full_spec((1, 1))],
    out_specs=_row_spec((BP, 1)),
    out_shape=jax.ShapeDtypeStruct((P, 1), jnp.float32),
)


def kernel(x, edge_index, edge_pairs, W1, b1, W2, b2, W3, b3, Wp1, bp1, Wp2, bp2):
    src = edge_index[0]
    dst = edge_index[1]
    npad = E_PAD - E
    pad_i = jnp.arange(npad, dtype=jnp.int32)
    srcw = jnp.concatenate([src, (pad_i * 997) % N]).reshape(NW, G, CH)
    dstw = jnp.concatenate([dst, N + (pad_i % 16)]).reshape(NW, G, CH)
    idxw = jnp.concatenate([edge_pairs[0], edge_pairs[1]]).reshape(NW, PG, CH)

    zeros = jnp.zeros((ACC_ROWS, H), jnp.float32)
    ones = jnp.ones((CH, H), jnp.float32)

    deg_parts = _deg(dstw, ones, zeros)
    hw1 = _tc_mm(x, W1)
    dis, dinv, hp1 = _tc_sp(deg_parts, hw1)
    s1 = _agg(hp1, srcw, dstw, zeros)
    hw2, hp2 = _tc_mid(s1, hw1, dis, dinv, b1.reshape(1, H), W2)
    s2 = _agg(hp2, srcw, dstw, zeros)
    hw3, hp3 = _tc_mid(s2, hw2, dis, dinv, b2.reshape(1, H), W3)
    s3 = _agg(hp3, srcw, dstw, zeros)
    h3 = _tc_fin(s3, hw3, dis, dinv, b3.reshape(1, H))

    g = _pairs(h3, idxw)
    pred = _tc_pred(g[:P], g[P:], Wp1[:H], Wp1[H:], bp1.reshape(1, H),
                    Wp2.reshape(1, H), bp2.reshape(1, 1))
    return pred.reshape(P)
